# baseline (speedup 1.0000x reference)
import functools

import jax
import jax.numpy as jnp
from jax import lax
from jax.experimental import pallas as pl
from jax.experimental.pallas import tpu as pltpu
from jax.experimental.pallas import tpu_sc as plsc

D = 32
LANES = 16
NC = 2
NS = 16
NW = NC * NS
CHUNK = 128


def _dwe_body(i_hbm, j_hbm, de_hbm, emb_hbm, out_hbm,
              i_v, j_v, de_v, u_v, v_v, o_v, sem, bpw):
    nchunks = bpw // CHUNK
    wid = lax.axis_index("s") * NC + lax.axis_index("c")
    base = wid * nchunks

    pltpu.sync_copy(i_hbm.at[pl.ds(base, nchunks)], i_v)
    pltpu.sync_copy(j_hbm.at[pl.ds(base, nchunks)], j_v)
    pltpu.sync_copy(de_hbm.at[pl.ds(base, nchunks)], de_v)

    copies = []
    for k in range(nchunks):
        copies.append(pltpu.async_copy(
            emb_hbm.at[i_v.at[k]], u_v.at[pl.ds(k * CHUNK, CHUNK)], sem))
        copies.append(pltpu.async_copy(
            emb_hbm.at[j_v.at[k]], v_v.at[pl.ds(k * CHUNK, CHUNK)], sem))
    for c in copies:
        c.wait()

    lane_iota = lax.broadcasted_iota(jnp.int32, (LANES,), 0)

    def group(g, carry):
        rows = lane_iota + g * LANES
        acc = jnp.zeros((LANES,), jnp.float32)
        for d in range(D):
            col = jnp.full((LANES,), d, jnp.int32)
            ud = plsc.load_gather(u_v, [rows, col])
            vd = plsc.load_gather(v_v, [rows, col])
            acc = acc + ud * vd
        dev = plsc.load_gather(de_v, [rows // CHUNK, rows % CHUNK])
        x = dev * acc
        s = 1.0 / (1.0 + jnp.exp(-x))
        o_v[pl.ds(g * LANES, LANES)] = -s
        return carry

    lax.fori_loop(0, bpw // LANES, group, 0)

    pltpu.sync_copy(o_v, out_hbm.at[pl.ds(wid * bpw, bpw)])


def kernel(pair, emb):
    B = pair.shape[0]
    bpw = B // NW
    i = pair[:, 0].astype(jnp.int32).reshape(B // CHUNK, CHUNK)
    j = pair[:, 1].astype(jnp.int32).reshape(B // CHUNK, CHUNK)
    de = pair[:, 2].astype(jnp.float32).reshape(B // CHUNK, CHUNK)

    mesh = plsc.VectorSubcoreMesh(core_axis_name="c", subcore_axis_name="s")
    run = pl.kernel(
        functools.partial(_dwe_body, bpw=bpw),
        out_type=jax.ShapeDtypeStruct((B,), jnp.float32),
        mesh=mesh,
        compiler_params=pltpu.CompilerParams(
            needs_layout_passes=False, use_tc_tiling_on_sc=False),
        scratch_types=[
            pltpu.VMEM((bpw // CHUNK, CHUNK), jnp.int32),
            pltpu.VMEM((bpw // CHUNK, CHUNK), jnp.int32),
            pltpu.VMEM((bpw // CHUNK, CHUNK), jnp.float32),
            pltpu.VMEM((bpw, D), jnp.float32),
            pltpu.VMEM((bpw, D), jnp.float32),
            pltpu.VMEM((bpw,), jnp.float32),
            pltpu.SemaphoreType.DMA,
        ],
    )
    out = run(i, j, de, emb)
    return out.reshape(B, 1)
